# baseline (speedup 1.0000x reference)
import functools

import jax
import jax.numpy as jnp
from jax.experimental import pallas as pl
from jax.experimental.pallas import tpu as pltpu

_TILE = 128
_CHUNK = 1024
_KERNEL_NUM = 5
_LOG2E = 1.4426950408889634


def _prep_kernel(src_ref, tgt_ref, tot_ref, rr_ref, scal_ref, *, n, b):
    s = src_ref[...]
    t = tgt_ref[...]
    mean = (jnp.sum(s, axis=0, keepdims=True)
            + jnp.sum(t, axis=0, keepdims=True)) * (1.0 / n)
    sb = (s - mean).astype(jnp.bfloat16)
    tb = (t - mean).astype(jnp.bfloat16)
    tot_ref[:b] = sb
    tot_ref[b:] = tb
    sf = sb.astype(jnp.float32)
    tf = tb.astype(jnp.float32)
    sq_s = jnp.sum(sf * sf, axis=1, keepdims=True)
    sq_t = jnp.sum(tf * tf, axis=1, keepdims=True)
    col = jnp.sum(sf, axis=0, keepdims=True) + jnp.sum(tf, axis=0, keepdims=True)
    ssq = jnp.sum(sq_s) + jnp.sum(sq_t)
    sum_l2 = 2.0 * n * ssq - 2.0 * jnp.sum(col * col)
    bandwidth = sum_l2 / float(n * n - n)
    c4 = -_LOG2E / (4.0 * bandwidth)
    rr_ref[:b] = sq_s * c4
    rr_ref[b:] = sq_t * c4
    scal_ref[0, 0] = -2.0 * c4


def _strip_kernel(scal_ref, xr_ref, tot_ref, rr_ref, rc_ref, out_ref, *,
                  n, chunk):
    m2 = scal_ref[0, 0]
    xr = (xr_ref[...].astype(jnp.float32) * m2).astype(jnp.bfloat16)
    rr = rr_ref[...]
    for c in range(n // chunk):
        sl = pl.ds(c * chunk, chunk)
        gram = jax.lax.dot_general(
            xr, tot_ref[sl, :],
            (((1,), (1,)), ((), ())),
            preferred_element_type=jnp.float32)
        t = gram + (rr + rc_ref[:, sl])
        e = jnp.exp2(t)
        acc = e
        for _ in range(_KERNEL_NUM - 1):
            e = e * e
            acc = acc + e
        out_ref[:, sl] = acc


def kernel(source, target):
    b, d = int(source.shape[0]), int(source.shape[1])
    n = b + int(target.shape[0])

    tot_bf, rr, scal = pl.pallas_call(
        functools.partial(_prep_kernel, n=n, b=b),
        out_shape=(
            jax.ShapeDtypeStruct((n, d), jnp.bfloat16),
            jax.ShapeDtypeStruct((n, 1), jnp.float32),
            jax.ShapeDtypeStruct((1, 1), jnp.float32),
        ),
        out_specs=(
            pl.BlockSpec(memory_space=pltpu.MemorySpace.VMEM),
            pl.BlockSpec(memory_space=pltpu.MemorySpace.VMEM),
            pl.BlockSpec(memory_space=pltpu.MemorySpace.SMEM),
        ),
        compiler_params=pltpu.CompilerParams(
            vmem_limit_bytes=100 * 1024 * 1024),
    )(source, target)
    rc = rr.reshape(1, n)

    grid = (n // _TILE,)
    body = functools.partial(_strip_kernel, n=n, chunk=_CHUNK)
    cost = pl.CostEstimate(
        flops=int(2 * n * n * d + 11 * n * n),
        transcendentals=int(n * n),
        bytes_accessed=int(2 * n * d * 2 + n * n * 4))
    out = pl.pallas_call(
        body,
        out_shape=jax.ShapeDtypeStruct((n, n), jnp.float32),
        grid=grid,
        in_specs=[
            pl.BlockSpec(memory_space=pltpu.MemorySpace.SMEM),
            pl.BlockSpec((_TILE, d), lambda i: (i, 0)),
            pl.BlockSpec((n, d), lambda i: (0, 0)),
            pl.BlockSpec((_TILE, 1), lambda i: (i, 0)),
            pl.BlockSpec((1, n), lambda i: (0, 0)),
        ],
        out_specs=pl.BlockSpec((_TILE, n), lambda i: (i, 0)),
        compiler_params=pltpu.CompilerParams(
            dimension_semantics=("parallel",),
            vmem_limit_bytes=100 * 1024 * 1024),
        cost_estimate=cost,
    )(scal, tot_bf, tot_bf, rr, rc)
    return out


# speedup vs baseline: 1.5079x; 1.5079x over previous
import functools

import jax
import jax.numpy as jnp
from jax.experimental import pallas as pl
from jax.experimental.pallas import tpu as pltpu

_TILE = 256
_CHUNK = 2048
_KERNEL_NUM = 5
_LOG2E = 1.4426950408889634


def _prep_kernel(src_ref, tgt_ref, tot_ref, rr_ref, scal_ref, *, n, b):
    s = src_ref[...]
    t = tgt_ref[...]
    mean = (jnp.sum(s, axis=0, keepdims=True)
            + jnp.sum(t, axis=0, keepdims=True)) * (1.0 / n)
    sb = (s - mean).astype(jnp.bfloat16)
    tb = (t - mean).astype(jnp.bfloat16)
    tot_ref[:b] = sb
    tot_ref[b:] = tb
    sf = sb.astype(jnp.float32)
    tf = tb.astype(jnp.float32)
    sq_s = jnp.sum(sf * sf, axis=1, keepdims=True)
    sq_t = jnp.sum(tf * tf, axis=1, keepdims=True)
    col = jnp.sum(sf, axis=0, keepdims=True) + jnp.sum(tf, axis=0, keepdims=True)
    ssq = jnp.sum(sq_s) + jnp.sum(sq_t)
    sum_l2 = 2.0 * n * ssq - 2.0 * jnp.sum(col * col)
    bandwidth = sum_l2 / float(n * n - n)
    c4 = -_LOG2E / (4.0 * bandwidth)
    rr_ref[:b] = sq_s * c4
    rr_ref[b:] = sq_t * c4
    scal_ref[0, 0] = -2.0 * c4


def _strip_kernel(scal_ref, xr_ref, tot_ref, rr_ref, rc_ref, out_ref, *,
                  n, chunk):
    m2 = scal_ref[0, 0]
    xr = (xr_ref[...].astype(jnp.float32) * m2).astype(jnp.bfloat16)
    rr = rr_ref[...]
    for c in range(n // chunk):
        sl = pl.ds(c * chunk, chunk)
        gram = jax.lax.dot_general(
            xr, tot_ref[sl, :],
            (((1,), (1,)), ((), ())),
            preferred_element_type=jnp.float32)
        t = gram + (rr + rc_ref[:, sl])
        e = jnp.exp2(t)
        acc = e
        for _ in range(_KERNEL_NUM - 1):
            e = e * e
            acc = acc + e
        out_ref[:, sl] = acc


def kernel(source, target):
    b, d = int(source.shape[0]), int(source.shape[1])
    n = b + int(target.shape[0])

    tot_bf, rr, scal = pl.pallas_call(
        functools.partial(_prep_kernel, n=n, b=b),
        out_shape=(
            jax.ShapeDtypeStruct((n, d), jnp.bfloat16),
            jax.ShapeDtypeStruct((n, 1), jnp.float32),
            jax.ShapeDtypeStruct((1, 1), jnp.float32),
        ),
        out_specs=(
            pl.BlockSpec(memory_space=pltpu.MemorySpace.VMEM),
            pl.BlockSpec(memory_space=pltpu.MemorySpace.VMEM),
            pl.BlockSpec(memory_space=pltpu.MemorySpace.SMEM),
        ),
        compiler_params=pltpu.CompilerParams(
            vmem_limit_bytes=100 * 1024 * 1024),
    )(source, target)
    rc = rr.reshape(1, n)

    grid = (n // _TILE,)
    body = functools.partial(_strip_kernel, n=n, chunk=_CHUNK)
    cost = pl.CostEstimate(
        flops=int(2 * n * n * d + 11 * n * n),
        transcendentals=int(n * n),
        bytes_accessed=int(2 * n * d * 2 + n * n * 4))
    out = pl.pallas_call(
        body,
        out_shape=jax.ShapeDtypeStruct((n, n), jnp.float32),
        grid=grid,
        in_specs=[
            pl.BlockSpec(memory_space=pltpu.MemorySpace.SMEM),
            pl.BlockSpec((_TILE, d), lambda i: (i, 0)),
            pl.BlockSpec((n, d), lambda i: (0, 0)),
            pl.BlockSpec((_TILE, 1), lambda i: (i, 0)),
            pl.BlockSpec((1, n), lambda i: (0, 0)),
        ],
        out_specs=pl.BlockSpec((_TILE, n), lambda i: (i, 0)),
        compiler_params=pltpu.CompilerParams(
            dimension_semantics=("parallel",),
            vmem_limit_bytes=100 * 1024 * 1024),
        cost_estimate=cost,
    )(scal, tot_bf, tot_bf, rr, rc)
    return out


# speedup vs baseline: 1.5103x; 1.0016x over previous
import functools

import jax
import jax.numpy as jnp
from jax.experimental import pallas as pl
from jax.experimental.pallas import tpu as pltpu

_TILE = 256
_CHUNK = 512
_KERNEL_NUM = 5
_LOG2E = 1.4426950408889634


def _prep_kernel(src_ref, tgt_ref, tot_ref, rr_ref, scal_ref, *, n, b):
    s = src_ref[...]
    t = tgt_ref[...]
    mean = (jnp.sum(s, axis=0, keepdims=True)
            + jnp.sum(t, axis=0, keepdims=True)) * (1.0 / n)
    sb = (s - mean).astype(jnp.bfloat16)
    tb = (t - mean).astype(jnp.bfloat16)
    tot_ref[:b] = sb
    tot_ref[b:] = tb
    sf = sb.astype(jnp.float32)
    tf = tb.astype(jnp.float32)
    sq_s = jnp.sum(sf * sf, axis=1, keepdims=True)
    sq_t = jnp.sum(tf * tf, axis=1, keepdims=True)
    col = jnp.sum(sf, axis=0, keepdims=True) + jnp.sum(tf, axis=0, keepdims=True)
    ssq = jnp.sum(sq_s) + jnp.sum(sq_t)
    sum_l2 = 2.0 * n * ssq - 2.0 * jnp.sum(col * col)
    bandwidth = sum_l2 / float(n * n - n)
    c4 = -_LOG2E / (4.0 * bandwidth)
    rr_ref[:b] = sq_s * c4
    rr_ref[b:] = sq_t * c4
    scal_ref[0, 0] = -2.0 * c4


def _strip_kernel(scal_ref, xr_ref, tot_ref, rr_ref, rc_ref, out_ref, *,
                  n, chunk):
    m2 = scal_ref[0, 0]
    xr = (xr_ref[...].astype(jnp.float32) * m2).astype(jnp.bfloat16)
    rr = rr_ref[...]
    for c in range(n // chunk):
        sl = pl.ds(c * chunk, chunk)
        gram = jax.lax.dot_general(
            xr, tot_ref[sl, :],
            (((1,), (1,)), ((), ())),
            preferred_element_type=jnp.float32)
        t = gram + (rr + rc_ref[:, sl])
        e = jnp.exp2(t)
        acc = e
        for _ in range(_KERNEL_NUM - 1):
            e = e * e
            acc = acc + e
        out_ref[:, sl] = acc


def kernel(source, target):
    b, d = int(source.shape[0]), int(source.shape[1])
    n = b + int(target.shape[0])

    tot_bf, rr, scal = pl.pallas_call(
        functools.partial(_prep_kernel, n=n, b=b),
        out_shape=(
            jax.ShapeDtypeStruct((n, d), jnp.bfloat16),
            jax.ShapeDtypeStruct((n, 1), jnp.float32),
            jax.ShapeDtypeStruct((1, 1), jnp.float32),
        ),
        out_specs=(
            pl.BlockSpec(memory_space=pltpu.MemorySpace.VMEM),
            pl.BlockSpec(memory_space=pltpu.MemorySpace.VMEM),
            pl.BlockSpec(memory_space=pltpu.MemorySpace.SMEM),
        ),
        compiler_params=pltpu.CompilerParams(
            vmem_limit_bytes=100 * 1024 * 1024),
    )(source, target)
    rc = rr.reshape(1, n)

    grid = (n // _TILE,)
    body = functools.partial(_strip_kernel, n=n, chunk=_CHUNK)
    cost = pl.CostEstimate(
        flops=int(2 * n * n * d + 11 * n * n),
        transcendentals=int(n * n),
        bytes_accessed=int(2 * n * d * 2 + n * n * 4))
    out = pl.pallas_call(
        body,
        out_shape=jax.ShapeDtypeStruct((n, n), jnp.float32),
        grid=grid,
        in_specs=[
            pl.BlockSpec(memory_space=pltpu.MemorySpace.SMEM),
            pl.BlockSpec((_TILE, d), lambda i: (i, 0)),
            pl.BlockSpec((n, d), lambda i: (0, 0)),
            pl.BlockSpec((_TILE, 1), lambda i: (i, 0)),
            pl.BlockSpec((1, n), lambda i: (0, 0)),
        ],
        out_specs=pl.BlockSpec((_TILE, n), lambda i: (i, 0)),
        compiler_params=pltpu.CompilerParams(
            dimension_semantics=("parallel",),
            vmem_limit_bytes=100 * 1024 * 1024),
        cost_estimate=cost,
    )(scal, tot_bf, tot_bf, rr, rc)
    return out


# speedup vs baseline: 1.5141x; 1.0025x over previous
import functools

import jax
import jax.numpy as jnp
from jax.experimental import pallas as pl
from jax.experimental.pallas import tpu as pltpu

_TILE = 256
_CHUNK = 512
_KERNEL_NUM = 5
_LOG2E = 1.4426950408889634


def _prep_kernel(src_ref, tgt_ref, tot_ref, rr_ref, scal_ref, *, n, b):
    s = src_ref[...]
    t = tgt_ref[...]
    mean = (jnp.sum(s, axis=0, keepdims=True)
            + jnp.sum(t, axis=0, keepdims=True)) * (1.0 / n)
    sb = (s - mean).astype(jnp.bfloat16)
    tb = (t - mean).astype(jnp.bfloat16)
    tot_ref[:b] = sb
    tot_ref[b:] = tb
    sf = sb.astype(jnp.float32)
    tf = tb.astype(jnp.float32)
    sq_s = jnp.sum(sf * sf, axis=1, keepdims=True)
    sq_t = jnp.sum(tf * tf, axis=1, keepdims=True)
    col = jnp.sum(sf, axis=0, keepdims=True) + jnp.sum(tf, axis=0, keepdims=True)
    ssq = jnp.sum(sq_s) + jnp.sum(sq_t)
    sum_l2 = 2.0 * n * ssq - 2.0 * jnp.sum(col * col)
    bandwidth = sum_l2 / float(n * n - n)
    c4 = -_LOG2E / (4.0 * bandwidth)
    rr_ref[:b] = sq_s * c4
    rr_ref[b:] = sq_t * c4
    scal_ref[0, 0] = -2.0 * c4


def _strip_kernel(scal_ref, tot_ref, rr_ref, rc_ref, out_ref, *,
                  n, chunk, tile):
    m2 = scal_ref[0, 0]
    i = pl.program_id(0)
    xr = (tot_ref[pl.ds(i * tile, tile), :].astype(jnp.float32)
          * m2).astype(jnp.bfloat16)
    rr = rr_ref[...]
    for c in range(n // chunk):
        sl = pl.ds(c * chunk, chunk)
        gram = jax.lax.dot_general(
            xr, tot_ref[sl, :],
            (((1,), (1,)), ((), ())),
            preferred_element_type=jnp.float32)
        t = gram + (rr + rc_ref[:, sl])
        e = jnp.exp2(t)
        acc = e
        for _ in range(_KERNEL_NUM - 1):
            e = e * e
            acc = acc + e
        out_ref[:, sl] = acc


def kernel(source, target):
    b, d = int(source.shape[0]), int(source.shape[1])
    n = b + int(target.shape[0])

    tot_bf, rr, scal = pl.pallas_call(
        functools.partial(_prep_kernel, n=n, b=b),
        out_shape=(
            jax.ShapeDtypeStruct((n, d), jnp.bfloat16),
            jax.ShapeDtypeStruct((n, 1), jnp.float32),
            jax.ShapeDtypeStruct((1, 1), jnp.float32),
        ),
        out_specs=(
            pl.BlockSpec(memory_space=pltpu.MemorySpace.VMEM),
            pl.BlockSpec(memory_space=pltpu.MemorySpace.VMEM),
            pl.BlockSpec(memory_space=pltpu.MemorySpace.SMEM),
        ),
        compiler_params=pltpu.CompilerParams(
            vmem_limit_bytes=100 * 1024 * 1024),
    )(source, target)
    rc = rr.reshape(1, n)

    grid = (n // _TILE,)
    body = functools.partial(_strip_kernel, n=n, chunk=_CHUNK, tile=_TILE)
    cost = pl.CostEstimate(
        flops=int(2 * n * n * d + 11 * n * n),
        transcendentals=int(n * n),
        bytes_accessed=int(2 * n * d * 2 + n * n * 4))
    out = pl.pallas_call(
        body,
        out_shape=jax.ShapeDtypeStruct((n, n), jnp.float32),
        grid=grid,
        in_specs=[
            pl.BlockSpec(memory_space=pltpu.MemorySpace.SMEM),
            pl.BlockSpec((n, d), lambda i: (0, 0)),
            pl.BlockSpec((_TILE, 1), lambda i: (i, 0)),
            pl.BlockSpec((1, n), lambda i: (0, 0)),
        ],
        out_specs=pl.BlockSpec((_TILE, n), lambda i: (i, 0)),
        compiler_params=pltpu.CompilerParams(
            dimension_semantics=("parallel",),
            vmem_limit_bytes=100 * 1024 * 1024),
        cost_estimate=cost,
    )(scal, tot_bf, rr, rc)
    return out


# speedup vs baseline: 1.5187x; 1.0031x over previous
import functools

import jax
import jax.numpy as jnp
from jax.experimental import pallas as pl
from jax.experimental.pallas import tpu as pltpu

_TILE = 256
_CHUNK = 512
_KERNEL_NUM = 5
_LOG2E = 1.4426950408889634


def _prep_kernel(src_ref, tgt_ref, tot_ref, rr_ref, scal_ref, *, n, b):
    s = src_ref[...]
    t = tgt_ref[...]
    mean = (jnp.sum(s, axis=0, keepdims=True)
            + jnp.sum(t, axis=0, keepdims=True)) * (1.0 / n)
    sb = (s - mean).astype(jnp.bfloat16)
    tb = (t - mean).astype(jnp.bfloat16)
    tot_ref[:b] = sb
    tot_ref[b:] = tb
    sf = sb.astype(jnp.float32)
    tf = tb.astype(jnp.float32)
    sq_s = jnp.sum(sf * sf, axis=1, keepdims=True)
    sq_t = jnp.sum(tf * tf, axis=1, keepdims=True)
    col = jnp.sum(sf, axis=0, keepdims=True) + jnp.sum(tf, axis=0, keepdims=True)
    ssq = jnp.sum(sq_s) + jnp.sum(sq_t)
    sum_l2 = 2.0 * n * ssq - 2.0 * jnp.sum(col * col)
    bandwidth = sum_l2 / float(n * n - n)
    c4 = -_LOG2E / (4.0 * bandwidth)
    rr_ref[:b] = sq_s * c4
    rr_ref[b:] = sq_t * c4
    scal_ref[0, 0] = -2.0 * c4


def _strip_kernel(scal_ref, tot_ref, rr_ref, rc_ref, out_ref, *,
                  n, chunk, tile):
    m2 = scal_ref[0, 0]
    i = pl.program_id(0)
    xr = (tot_ref[pl.ds(i * tile, tile), :].astype(jnp.float32)
          * m2).astype(jnp.bfloat16)
    rr = rr_ref[...]
    for c in range(n // chunk):
        sl = pl.ds(c * chunk, chunk)
        gram = jax.lax.dot_general(
            xr, tot_ref[sl, :],
            (((1,), (1,)), ((), ())),
            preferred_element_type=jnp.float32)
        t = gram + (rr + rc_ref[:, sl])
        e = jnp.exp2(t)
        acc = e
        for _ in range(_KERNEL_NUM - 1):
            e = e * e
            acc = acc + e
        out_ref[:, sl] = acc


def kernel(source, target):
    b, d = int(source.shape[0]), int(source.shape[1])
    n = b + int(target.shape[0])

    tot_bf, rr, scal = pl.pallas_call(
        functools.partial(_prep_kernel, n=n, b=b),
        out_shape=(
            jax.ShapeDtypeStruct((n, d), jnp.bfloat16),
            jax.ShapeDtypeStruct((n, 1), jnp.float32),
            jax.ShapeDtypeStruct((1, 1), jnp.float32),
        ),
        out_specs=(
            pl.BlockSpec(memory_space=pltpu.MemorySpace.VMEM),
            pl.BlockSpec(memory_space=pltpu.MemorySpace.VMEM),
            pl.BlockSpec(memory_space=pltpu.MemorySpace.SMEM),
        ),
        compiler_params=pltpu.CompilerParams(
            vmem_limit_bytes=100 * 1024 * 1024),
    )(source, target)
    rc = rr.reshape(1, n)

    grid = (n // _TILE,)
    body = functools.partial(_strip_kernel, n=n, chunk=_CHUNK, tile=_TILE)
    cost = pl.CostEstimate(
        flops=int(2 * n * n * d + 11 * n * n),
        transcendentals=int(n * n),
        bytes_accessed=int(2 * n * d * 2 + n * n * 4))
    out = pl.pallas_call(
        body,
        out_shape=jax.ShapeDtypeStruct((n, n), jnp.float32),
        grid=grid,
        in_specs=[
            pl.BlockSpec(memory_space=pltpu.MemorySpace.SMEM),
            pl.BlockSpec((n, d), lambda i: (0, 0)),
            pl.BlockSpec((_TILE, 1), lambda i: (i, 0)),
            pl.BlockSpec((1, n), lambda i: (0, 0)),
        ],
        out_specs=pl.BlockSpec((_TILE, n), lambda i: (i, 0)),
        compiler_params=pltpu.CompilerParams(
            dimension_semantics=("arbitrary",),
            vmem_limit_bytes=100 * 1024 * 1024),
        cost_estimate=cost,
    )(scal, tot_bf, rr, rc)
    return out
